# Initial kernel scaffold; baseline (speedup 1.0000x reference)
import functools

import jax
import jax.numpy as jnp
from jax import lax
from jax.experimental import pallas as pl
from jax.experimental.pallas import tpu as pltpu
from jax.experimental.pallas import tpu_sc as plsc

NN = 10000
EE = 160000
NPAD = 10240
DIN = 256
HIDW = 128
NHEADS = 4
TW = 144
ADW = 16
BLK = 256
NBLK = NPAD // BLK

NTILE = 16
NCORE = 2
ROWS_PER_TILE = NPAD // NTILE
EB1 = 80
NB1 = (EE // NTILE) // EB1
EB2 = 40
NB2 = (EE // (NTILE * NCORE)) // EB2


def _l1_tables_body(x_ref, w1_ref, a1s_ref, a1d_ref, t_ref, ad_ref):
    h = jnp.dot(x_ref[...], w1_ref[...], preferred_element_type=jnp.float32)
    ad_ref[:, NHEADS:ADW] = jnp.zeros((BLK, ADW - NHEADS), jnp.float32)
    for hd in range(NHEADS):
        hh = h[:, hd * HIDW:(hd + 1) * HIDW]
        asv = jnp.sum(hh * a1s_ref[hd, :][None, :], axis=1, keepdims=True)
        adv = jnp.sum(hh * a1d_ref[hd, :][None, :], axis=1, keepdims=True)
        t_ref[hd, :, 0:HIDW] = hh
        t_ref[hd, :, HIDW:HIDW + 1] = jnp.ones((BLK, 1), jnp.float32)
        t_ref[hd, :, HIDW + 1:HIDW + 2] = asv
        t_ref[hd, :, HIDW + 2:TW] = jnp.zeros((BLK, TW - HIDW - 2), jnp.float32)
        ad_ref[:, hd:hd + 1] = adv


def _l1_tables(x_pad, W1, a1_src, a1_dst):
    return pl.pallas_call(
        _l1_tables_body,
        grid=(NBLK,),
        in_specs=[
            pl.BlockSpec((BLK, DIN), lambda i: (i, 0)),
            pl.BlockSpec((DIN, NHEADS * HIDW), lambda i: (0, 0)),
            pl.BlockSpec((NHEADS, HIDW), lambda i: (0, 0)),
            pl.BlockSpec((NHEADS, HIDW), lambda i: (0, 0)),
        ],
        out_specs=[
            pl.BlockSpec((NHEADS, BLK, TW), lambda i: (0, i, 0)),
            pl.BlockSpec((BLK, ADW), lambda i: (i, 0)),
        ],
        out_shape=[
            jax.ShapeDtypeStruct((NHEADS, NPAD, TW), jnp.float32),
            jax.ShapeDtypeStruct((NPAD, ADW), jnp.float32),
        ],
    )(x_pad, W1, a1_src, a1_dst)


def _l2_tables_body(acc_ref, b1_ref, w2_ref, a2s_ref, a2d_ref, t_ref, ad_ref):
    hs = []
    for hd in range(NHEADS):
        num = acc_ref[hd, :, 0:HIDW]
        den = acc_ref[hd, :, HIDW:HIDW + 1]
        v = num / (den + 1e-16) + b1_ref[hd, :][None, :]
        hs.append(jnp.where(v > 0, v, jnp.expm1(v)))
    h1n = jnp.concatenate(hs, axis=1)
    h2 = jnp.dot(h1n, w2_ref[...], preferred_element_type=jnp.float32)
    asv = jnp.sum(h2 * a2s_ref[0, :][None, :], axis=1, keepdims=True)
    adv = jnp.sum(h2 * a2d_ref[0, :][None, :], axis=1, keepdims=True)
    t_ref[:, 0:HIDW] = h2
    t_ref[:, HIDW:HIDW + 1] = jnp.ones((BLK, 1), jnp.float32)
    t_ref[:, HIDW + 1:HIDW + 2] = asv
    t_ref[:, HIDW + 2:TW] = jnp.zeros((BLK, TW - HIDW - 2), jnp.float32)
    ad_ref[:, 0:1] = adv
    ad_ref[:, 1:ADW] = jnp.zeros((BLK, ADW - 1), jnp.float32)


def _l2_tables(acc1, b1r, W2, a2_src, a2_dst):
    return pl.pallas_call(
        _l2_tables_body,
        grid=(NBLK,),
        in_specs=[
            pl.BlockSpec((NHEADS, BLK, TW), lambda i: (0, i, 0)),
            pl.BlockSpec((NHEADS, HIDW), lambda i: (0, 0)),
            pl.BlockSpec((NHEADS * HIDW, HIDW), lambda i: (0, 0)),
            pl.BlockSpec((1, HIDW), lambda i: (0, 0)),
            pl.BlockSpec((1, HIDW), lambda i: (0, 0)),
        ],
        out_specs=[
            pl.BlockSpec((BLK, TW), lambda i: (i, 0)),
            pl.BlockSpec((BLK, ADW), lambda i: (i, 0)),
        ],
        out_shape=[
            jax.ShapeDtypeStruct((NPAD, TW), jnp.float32),
            jax.ShapeDtypeStruct((NPAD, ADW), jnp.float32),
        ],
    )(acc1, b1r, W2, a2_src, a2_dst)


def _final_body(acc_ref, b2_ref, wout_ref, bout_ref, o_ref):
    a = acc_ref[0] + acc_ref[1]
    v = a[:, 0:HIDW] / (a[:, HIDW:HIDW + 1] + 1e-16) + b2_ref[0, :][None, :]
    h = jnp.where(v > 0, v, jnp.expm1(v))
    o_ref[...] = jnp.dot(h, wout_ref[...],
                         preferred_element_type=jnp.float32) + bout_ref[0, :][None, :]


def _final(acc2, b2r, Wout, boutr):
    return pl.pallas_call(
        _final_body,
        grid=(NBLK,),
        in_specs=[
            pl.BlockSpec((NCORE, BLK, TW), lambda i: (0, i, 0)),
            pl.BlockSpec((1, HIDW), lambda i: (0, 0)),
            pl.BlockSpec((HIDW, 1), lambda i: (0, 0)),
            pl.BlockSpec((1, 1), lambda i: (0, 0)),
        ],
        out_specs=pl.BlockSpec((BLK, 1), lambda i: (i, 0)),
        out_shape=jax.ShapeDtypeStruct((NPAD, 1), jnp.float32),
    )(acc2, b2r, Wout, boutr)


def _zero_fill(zero_v, nrows):
    zv = jnp.zeros((16,), jnp.float32)

    def zrow(j, _):
        for k in range(TW // 16):
            zero_v[j, pl.ds(k * 16, 16)] = zv
        return 0

    lax.fori_loop(0, nrows, zrow, 0)


def _edge_batch(t_hbm, ad_hbm, src_hbm, dst_hbm, acc_sp, src_v, dst_v,
                rows_v, adrows_v, w_v, sem, base, nb, row_off, adcol):
    i16 = lax.iota(jnp.int32, 16)
    c129 = i16 * 0 + (HIDW + 1)
    cad = i16 * 0 + adcol
    pltpu.sync_copy(src_hbm.at[pl.ds(base, nb)], src_v)
    pltpu.sync_copy(dst_hbm.at[pl.ds(base, nb)], dst_v)
    if row_off is not None:
        def adjf(k, _):
            src_v[pl.ds(k * 16, 16)] = src_v[pl.ds(k * 16, 16)] + row_off
            return 0
        lax.fori_loop(0, nb // 16, adjf, 0)
    pltpu.async_copy(t_hbm.at[src_v], rows_v, sem).wait()
    pltpu.async_copy(ad_hbm.at[dst_v], adrows_v, sem).wait()
    offs = list(range(0, nb - 15, 16))
    if offs[-1] != nb - 16:
        offs.append(nb - 16)
    for off in offs:
        rr = i16 + off
        asg = plsc.load_gather(rows_v, [rr, c129])
        adg = plsc.load_gather(adrows_v, [rr, cad])
        e = asg + adg
        e = jnp.maximum(e, e * 0.2)
        w_v[pl.ds(off, 16)] = jnp.exp(e)

    def scale(r, _):
        wr = w_v[r]
        for k in range(TW // 16):
            rows_v[r, pl.ds(k * 16, 16)] = rows_v[r, pl.ds(k * 16, 16)] * wr
        return 0

    lax.fori_loop(0, nb, scale, 0)
    pltpu.sync_copy(rows_v, acc_sp.at[dst_v], add=True)


def _acc_zero(acc_sp, zero_v, s):
    def zacc(j, _):
        pltpu.sync_copy(zero_v, acc_sp.at[pl.ds(s * ROWS_PER_TILE + j * 128, 128)])
        return 0
    lax.fori_loop(0, ROWS_PER_TILE // 128, zacc, 0)


def _acc_readout(acc_sp, stage_v, out_hbm, s, out_row_base):
    def rdout(j, _):
        rs = s * ROWS_PER_TILE + j * 128
        pltpu.sync_copy(acc_sp.at[pl.ds(rs, 128)], stage_v)
        pltpu.sync_copy(stage_v, out_hbm.at[pl.ds(out_row_base + rs, 128)])
        return 0
    lax.fori_loop(0, ROWS_PER_TILE // 128, rdout, 0)


def _sc_layer1(tflat, adtab, src, dst):
    mesh = plsc.VectorSubcoreMesh(core_axis_name="c", subcore_axis_name="s")

    @functools.partial(
        pl.kernel,
        out_type=jax.ShapeDtypeStruct((NHEADS * NPAD, TW), jnp.float32),
        mesh=mesh,
        scratch_types=[
            pltpu.VMEM_SHARED((NPAD, TW), jnp.float32),
            pltpu.VMEM((EB1,), jnp.int32),
            pltpu.VMEM((EB1,), jnp.int32),
            pltpu.VMEM((EB1, TW), jnp.float32),
            pltpu.VMEM((EB1, ADW), jnp.float32),
            pltpu.VMEM((EB1,), jnp.float32),
            pltpu.VMEM((128, TW), jnp.float32),
            pltpu.VMEM((128, TW), jnp.float32),
            pltpu.SemaphoreType.DMA,
        ],
    )
    def body(t_hbm, ad_hbm, src_hbm, dst_hbm, out_hbm, acc_sp, src_v, dst_v,
             rows_v, adrows_v, w_v, zero_v, stage_v, sem):
        c = lax.axis_index("c")
        s = lax.axis_index("s")
        _zero_fill(zero_v, 128)
        for p in range(2):
            hd = 2 * c + p
            _acc_zero(acc_sp, zero_v, s)
            plsc.subcore_barrier()

            def batch(i, _):
                _edge_batch(t_hbm, ad_hbm, src_hbm, dst_hbm, acc_sp, src_v,
                            dst_v, rows_v, adrows_v, w_v, sem,
                            s * (EE // NTILE) + i * EB1, EB1,
                            hd * NPAD, hd)
                return 0

            lax.fori_loop(0, NB1, batch, 0)
            plsc.subcore_barrier()
            _acc_readout(acc_sp, stage_v, out_hbm, s, hd * NPAD)
            plsc.subcore_barrier()

    return body(tflat, adtab, src, dst)


def _sc_layer2(tbl, adtab, src, dst):
    mesh = plsc.VectorSubcoreMesh(core_axis_name="c", subcore_axis_name="s")

    @functools.partial(
        pl.kernel,
        out_type=jax.ShapeDtypeStruct((NCORE * NPAD, TW), jnp.float32),
        mesh=mesh,
        scratch_types=[
            pltpu.VMEM_SHARED((NPAD, TW), jnp.float32),
            pltpu.VMEM((EB2,), jnp.int32),
            pltpu.VMEM((EB2,), jnp.int32),
            pltpu.VMEM((EB2, TW), jnp.float32),
            pltpu.VMEM((EB2, ADW), jnp.float32),
            pltpu.VMEM((EB2,), jnp.float32),
            pltpu.VMEM((128, TW), jnp.float32),
            pltpu.VMEM((128, TW), jnp.float32),
            pltpu.SemaphoreType.DMA,
        ],
    )
    def body(t_hbm, ad_hbm, src_hbm, dst_hbm, out_hbm, acc_sp, src_v, dst_v,
             rows_v, adrows_v, w_v, zero_v, stage_v, sem):
        c = lax.axis_index("c")
        s = lax.axis_index("s")
        wid = c * NTILE + s
        _zero_fill(zero_v, 128)
        _acc_zero(acc_sp, zero_v, s)
        plsc.subcore_barrier()

        def batch(i, _):
            _edge_batch(t_hbm, ad_hbm, src_hbm, dst_hbm, acc_sp, src_v,
                        dst_v, rows_v, adrows_v, w_v, sem,
                        wid * (EE // (NTILE * NCORE)) + i * EB2, EB2,
                        None, 0)
            return 0

        lax.fori_loop(0, NB2, batch, 0)
        plsc.subcore_barrier()
        _acc_readout(acc_sp, stage_v, out_hbm, s, c * NPAD)

    return body(tbl, adtab, src, dst)


def kernel(x, edge_index, W1, a1_src, a1_dst, b1, W2, a2_src, a2_dst, b2,
           Wout, bout):
    x_pad = jnp.pad(x, ((0, NPAD - NN), (0, 0)))
    src = edge_index[0]
    dst = edge_index[1]

    table1, adtab1 = _l1_tables(x_pad, W1, a1_src, a1_dst)
    acc1 = _sc_layer1(table1.reshape(NHEADS * NPAD, TW), adtab1, src, dst)
    acc1 = acc1.reshape(NHEADS, NPAD, TW)

    table2, adtab2 = _l2_tables(acc1, b1.reshape(NHEADS, HIDW), W2,
                                a2_src, a2_dst)
    acc2 = _sc_layer2(table2, adtab2, src, dst)
    acc2 = acc2.reshape(NCORE, NPAD, TW)

    y = _final(acc2, b2.reshape(1, HIDW), Wout, bout.reshape(1, 1))
    return y[:NN]


# speedup vs baseline: 15.2062x; 15.2062x over previous
import functools

import jax
import jax.numpy as jnp
from jax import lax
from jax.experimental import pallas as pl
from jax.experimental.pallas import tpu as pltpu
from jax.experimental.pallas import tpu_sc as plsc

NN = 10000
EE = 160000
NPAD = 10240
DIN = 256
HIDW = 128
NHEADS = 4
TW = 144
ADW = 16
BLK = 256
NBLK = NPAD // BLK

NTILE = 16
NCORE = 2
ROWS_PER_TILE = NPAD // NTILE
EB1 = 80
NB1 = (EE // NTILE) // EB1
EB2 = 40
NB2 = (EE // (NTILE * NCORE)) // EB2


def _l1_tables_body(x_ref, w1_ref, a1s_ref, a1d_ref, t_ref, ad_ref):
    h = jnp.dot(x_ref[...], w1_ref[...], preferred_element_type=jnp.float32)
    ad_ref[:, NHEADS:ADW] = jnp.zeros((BLK, ADW - NHEADS), jnp.float32)
    for hd in range(NHEADS):
        hh = h[:, hd * HIDW:(hd + 1) * HIDW]
        asv = jnp.sum(hh * a1s_ref[hd, :][None, :], axis=1, keepdims=True)
        adv = jnp.sum(hh * a1d_ref[hd, :][None, :], axis=1, keepdims=True)
        t_ref[hd, :, 0:HIDW] = hh
        t_ref[hd, :, HIDW:HIDW + 1] = jnp.ones((BLK, 1), jnp.float32)
        t_ref[hd, :, HIDW + 1:HIDW + 2] = asv
        t_ref[hd, :, HIDW + 2:TW] = jnp.zeros((BLK, TW - HIDW - 2), jnp.float32)
        ad_ref[:, hd:hd + 1] = adv


def _l1_tables(x_pad, W1, a1_src, a1_dst):
    return pl.pallas_call(
        _l1_tables_body,
        grid=(NBLK,),
        in_specs=[
            pl.BlockSpec((BLK, DIN), lambda i: (i, 0)),
            pl.BlockSpec((DIN, NHEADS * HIDW), lambda i: (0, 0)),
            pl.BlockSpec((NHEADS, HIDW), lambda i: (0, 0)),
            pl.BlockSpec((NHEADS, HIDW), lambda i: (0, 0)),
        ],
        out_specs=[
            pl.BlockSpec((NHEADS, BLK, TW), lambda i: (0, i, 0)),
            pl.BlockSpec((BLK, ADW), lambda i: (i, 0)),
        ],
        out_shape=[
            jax.ShapeDtypeStruct((NHEADS, NPAD, TW), jnp.float32),
            jax.ShapeDtypeStruct((NPAD, ADW), jnp.float32),
        ],
    )(x_pad, W1, a1_src, a1_dst)


def _l2_tables_body(acc_ref, b1_ref, w2_ref, a2s_ref, a2d_ref, t_ref, ad_ref):
    hs = []
    for hd in range(NHEADS):
        num = acc_ref[hd, :, 0:HIDW]
        den = acc_ref[hd, :, HIDW:HIDW + 1]
        v = num / (den + 1e-16) + b1_ref[hd, :][None, :]
        hs.append(jnp.where(v > 0, v, jnp.exp(v) - 1.0))
    h1n = jnp.concatenate(hs, axis=1)
    h2 = jnp.dot(h1n, w2_ref[...], preferred_element_type=jnp.float32)
    asv = jnp.sum(h2 * a2s_ref[0, :][None, :], axis=1, keepdims=True)
    adv = jnp.sum(h2 * a2d_ref[0, :][None, :], axis=1, keepdims=True)
    t_ref[:, 0:HIDW] = h2
    t_ref[:, HIDW:HIDW + 1] = jnp.ones((BLK, 1), jnp.float32)
    t_ref[:, HIDW + 1:HIDW + 2] = asv
    t_ref[:, HIDW + 2:TW] = jnp.zeros((BLK, TW - HIDW - 2), jnp.float32)
    ad_ref[:, 0:1] = adv
    ad_ref[:, 1:ADW] = jnp.zeros((BLK, ADW - 1), jnp.float32)


def _l2_tables(acc1, b1r, W2, a2_src, a2_dst):
    return pl.pallas_call(
        _l2_tables_body,
        grid=(NBLK,),
        in_specs=[
            pl.BlockSpec((NHEADS, BLK, TW), lambda i: (0, i, 0)),
            pl.BlockSpec((NHEADS, HIDW), lambda i: (0, 0)),
            pl.BlockSpec((NHEADS * HIDW, HIDW), lambda i: (0, 0)),
            pl.BlockSpec((1, HIDW), lambda i: (0, 0)),
            pl.BlockSpec((1, HIDW), lambda i: (0, 0)),
        ],
        out_specs=[
            pl.BlockSpec((BLK, TW), lambda i: (i, 0)),
            pl.BlockSpec((BLK, ADW), lambda i: (i, 0)),
        ],
        out_shape=[
            jax.ShapeDtypeStruct((NPAD, TW), jnp.float32),
            jax.ShapeDtypeStruct((NPAD, ADW), jnp.float32),
        ],
    )(acc1, b1r, W2, a2_src, a2_dst)


def _final_body(acc_ref, b2_ref, wout_ref, bout_ref, o_ref):
    a = acc_ref[0] + acc_ref[1]
    v = a[:, 0:HIDW] / (a[:, HIDW:HIDW + 1] + 1e-16) + b2_ref[0, :][None, :]
    h = jnp.where(v > 0, v, jnp.exp(v) - 1.0)
    o_ref[...] = jnp.dot(h, wout_ref[...],
                         preferred_element_type=jnp.float32) + bout_ref[0, :][None, :]


def _final(acc2, b2r, Wout, boutr):
    return pl.pallas_call(
        _final_body,
        grid=(NBLK,),
        in_specs=[
            pl.BlockSpec((NCORE, BLK, TW), lambda i: (0, i, 0)),
            pl.BlockSpec((1, HIDW), lambda i: (0, 0)),
            pl.BlockSpec((HIDW, 1), lambda i: (0, 0)),
            pl.BlockSpec((1, 1), lambda i: (0, 0)),
        ],
        out_specs=pl.BlockSpec((BLK, 1), lambda i: (i, 0)),
        out_shape=jax.ShapeDtypeStruct((NPAD, 1), jnp.float32),
    )(acc2, b2r, Wout, boutr)


def _zero_fill(zero_v, nrows):
    zv = jnp.zeros((16,), jnp.float32)

    def zrow(j, _):
        for k in range(TW // 16):
            zero_v[j, pl.ds(k * 16, 16)] = zv
        return 0

    lax.fori_loop(0, nrows, zrow, 0)


def _edge_batch(t_hbm, ad_hbm, src_hbm, dst_hbm, acc_sp, src_v, dst_v,
                rows_v, adrows_v, w_v, sem, base, nb, row_off, adcol):
    i16 = lax.iota(jnp.int32, 16)
    c129 = i16 * 0 + (HIDW + 1)
    cad = i16 * 0 + adcol
    pltpu.sync_copy(src_hbm.at[pl.ds(base, nb)], src_v)
    pltpu.sync_copy(dst_hbm.at[pl.ds(base, nb)], dst_v)
    if row_off is not None:
        def adjf(k, _):
            src_v[pl.ds(k * 16, 16)] = src_v[pl.ds(k * 16, 16)] + row_off
            return 0
        lax.fori_loop(0, nb // 16, adjf, 0)
    pltpu.async_copy(t_hbm.at[src_v], rows_v, sem).wait()
    pltpu.async_copy(ad_hbm.at[dst_v], adrows_v, sem).wait()
    offs = list(range(0, nb - 15, 16))
    if offs[-1] != nb - 16:
        offs.append(nb - 16)
    for off in offs:
        rr = i16 + off
        asg = plsc.load_gather(rows_v, [rr, c129])
        adg = plsc.load_gather(adrows_v, [rr, cad])
        e = asg + adg
        e = jnp.maximum(e, e * 0.2)
        w_v[pl.ds(off, 16)] = jnp.exp(e)

    def scale_group(g, _):
        w16 = w_v[pl.ds(g * 16, 16)]
        r0 = g * 16
        for j in range(16):
            wr = w16[j]
            for k in range(TW // 16):
                rows_v[r0 + j, pl.ds(k * 16, 16)] = (
                    rows_v[r0 + j, pl.ds(k * 16, 16)] * wr)
        return 0

    lax.fori_loop(0, nb // 16, scale_group, 0)
    rem = nb - (nb // 16) * 16
    if rem:
        r0 = (nb // 16) * 16
        w16 = w_v[pl.ds(r0, 16)]
        for j in range(rem):
            wr = w16[j]
            for k in range(TW // 16):
                rows_v[r0 + j, pl.ds(k * 16, 16)] = (
                    rows_v[r0 + j, pl.ds(k * 16, 16)] * wr)
    pltpu.sync_copy(rows_v, acc_sp.at[dst_v], add=True)


def _acc_zero(acc_sp, zero_v, s):
    def zacc(j, _):
        pltpu.sync_copy(zero_v, acc_sp.at[pl.ds(s * ROWS_PER_TILE + j * 64, 64)])
        return 0
    lax.fori_loop(0, ROWS_PER_TILE // 64, zacc, 0)


def _acc_readout(acc_sp, stage_v, out_hbm, s, out_row_base):
    def rdout(j, _):
        rs = s * ROWS_PER_TILE + j * 64
        pltpu.sync_copy(acc_sp.at[pl.ds(rs, 64)], stage_v)
        pltpu.sync_copy(stage_v, out_hbm.at[pl.ds(out_row_base + rs, 64)])
        return 0
    lax.fori_loop(0, ROWS_PER_TILE // 64, rdout, 0)


def _sc_layer1(tflat, adtab, src, dst):
    mesh = plsc.VectorSubcoreMesh(core_axis_name="c", subcore_axis_name="s")

    @functools.partial(
        pl.kernel,
        out_type=jax.ShapeDtypeStruct((NHEADS * NPAD, TW), jnp.float32),
        mesh=mesh,
        compiler_params=pltpu.CompilerParams(use_tc_tiling_on_sc=False, needs_layout_passes=False),
        scratch_types=[
            pltpu.VMEM_SHARED((NPAD, TW), jnp.float32),
            pltpu.VMEM((EB1,), jnp.int32),
            pltpu.VMEM((EB1,), jnp.int32),
            pltpu.VMEM((EB1, TW), jnp.float32),
            pltpu.VMEM((EB1, ADW), jnp.float32),
            pltpu.VMEM((EB1,), jnp.float32),
            pltpu.VMEM((64, TW), jnp.float32),
            pltpu.VMEM((64, TW), jnp.float32),
            pltpu.SemaphoreType.DMA,
        ],
    )
    def body(t_hbm, ad_hbm, src_hbm, dst_hbm, out_hbm, acc_sp, src_v, dst_v,
             rows_v, adrows_v, w_v, zero_v, stage_v, sem):
        c = lax.axis_index("c")
        s = lax.axis_index("s")
        _zero_fill(zero_v, 64)
        for p in range(2):
            hd = 2 * c + p
            _acc_zero(acc_sp, zero_v, s)
            plsc.subcore_barrier()

            def batch(i, _):
                _edge_batch(t_hbm, ad_hbm, src_hbm, dst_hbm, acc_sp, src_v,
                            dst_v, rows_v, adrows_v, w_v, sem,
                            s * (EE // NTILE) + i * EB1, EB1,
                            hd * NPAD, hd)
                return 0

            lax.fori_loop(0, NB1, batch, 0)
            plsc.subcore_barrier()
            _acc_readout(acc_sp, stage_v, out_hbm, s, hd * NPAD)
            plsc.subcore_barrier()

    return body(tflat, adtab, src, dst)


def _sc_layer2(tbl, adtab, src, dst):
    mesh = plsc.VectorSubcoreMesh(core_axis_name="c", subcore_axis_name="s")

    @functools.partial(
        pl.kernel,
        out_type=jax.ShapeDtypeStruct((NCORE * NPAD, TW), jnp.float32),
        mesh=mesh,
        compiler_params=pltpu.CompilerParams(use_tc_tiling_on_sc=False, needs_layout_passes=False),
        scratch_types=[
            pltpu.VMEM_SHARED((NPAD, TW), jnp.float32),
            pltpu.VMEM((EB2,), jnp.int32),
            pltpu.VMEM((EB2,), jnp.int32),
            pltpu.VMEM((EB2, TW), jnp.float32),
            pltpu.VMEM((EB2, ADW), jnp.float32),
            pltpu.VMEM((48,), jnp.float32),
            pltpu.VMEM((64, TW), jnp.float32),
            pltpu.VMEM((64, TW), jnp.float32),
            pltpu.SemaphoreType.DMA,
        ],
    )
    def body(t_hbm, ad_hbm, src_hbm, dst_hbm, out_hbm, acc_sp, src_v, dst_v,
             rows_v, adrows_v, w_v, zero_v, stage_v, sem):
        c = lax.axis_index("c")
        s = lax.axis_index("s")
        wid = c * NTILE + s
        _zero_fill(zero_v, 64)
        _acc_zero(acc_sp, zero_v, s)
        plsc.subcore_barrier()

        def batch(i, _):
            _edge_batch(t_hbm, ad_hbm, src_hbm, dst_hbm, acc_sp, src_v,
                        dst_v, rows_v, adrows_v, w_v, sem,
                        wid * (EE // (NTILE * NCORE)) + i * EB2, EB2,
                        None, 0)
            return 0

        lax.fori_loop(0, NB2, batch, 0)
        plsc.subcore_barrier()
        _acc_readout(acc_sp, stage_v, out_hbm, s, c * NPAD)

    return body(tbl, adtab, src, dst)


def kernel(x, edge_index, W1, a1_src, a1_dst, b1, W2, a2_src, a2_dst, b2,
           Wout, bout):
    x_pad = jnp.pad(x, ((0, NPAD - NN), (0, 0)))
    src = edge_index[0]
    dst = edge_index[1]

    table1, adtab1 = _l1_tables(x_pad, W1, a1_src, a1_dst)
    acc1 = _sc_layer1(table1.reshape(NHEADS * NPAD, TW), adtab1, src, dst)
    acc1 = acc1.reshape(NHEADS, NPAD, TW)

    table2, adtab2 = _l2_tables(acc1, b1.reshape(NHEADS, HIDW), W2,
                                a2_src, a2_dst)
    acc2 = _sc_layer2(table2, adtab2, src, dst)
    acc2 = acc2.reshape(NCORE, NPAD, TW)

    y = _final(acc2, b2.reshape(1, HIDW), Wout, bout.reshape(1, 1))
    return y[:NN]


# speedup vs baseline: 31.6062x; 2.0785x over previous
import functools

import jax
import jax.numpy as jnp
from jax import lax
from jax.experimental import pallas as pl
from jax.experimental.pallas import tpu as pltpu
from jax.experimental.pallas import tpu_sc as plsc

NN = 10000
EE = 160000
NPAD = 10240
DIN = 256
HIDW = 128
NHEADS = 4
TW = 144
ADW = 16
BLK = 256
NBLK = NPAD // BLK

NTILE = 16
NCORE = 2
ROWS_PER_TILE = NPAD // NTILE
EB1 = 80
NB1 = (EE // NTILE) // EB1
EB2 = 40
NB2 = (EE // (NTILE * NCORE)) // EB2
EIDX_PAD = EE + 4 * EB1


def _l1_tables_body(x_ref, w1_ref, a1s_ref, a1d_ref, t_ref, ad_ref):
    h = jnp.dot(x_ref[...], w1_ref[...], preferred_element_type=jnp.float32)
    ad_ref[:, NHEADS:ADW] = jnp.zeros((BLK, ADW - NHEADS), jnp.float32)
    for hd in range(NHEADS):
        hh = h[:, hd * HIDW:(hd + 1) * HIDW]
        asv = jnp.sum(hh * a1s_ref[hd, :][None, :], axis=1, keepdims=True)
        adv = jnp.sum(hh * a1d_ref[hd, :][None, :], axis=1, keepdims=True)
        t_ref[hd, :, 0:HIDW] = hh
        t_ref[hd, :, HIDW:HIDW + 1] = jnp.ones((BLK, 1), jnp.float32)
        t_ref[hd, :, HIDW + 1:HIDW + 2] = asv
        t_ref[hd, :, HIDW + 2:TW] = jnp.zeros((BLK, TW - HIDW - 2), jnp.float32)
        ad_ref[:, hd:hd + 1] = adv


def _l1_tables(x_pad, W1, a1_src, a1_dst):
    return pl.pallas_call(
        _l1_tables_body,
        grid=(NBLK,),
        in_specs=[
            pl.BlockSpec((BLK, DIN), lambda i: (i, 0)),
            pl.BlockSpec((DIN, NHEADS * HIDW), lambda i: (0, 0)),
            pl.BlockSpec((NHEADS, HIDW), lambda i: (0, 0)),
            pl.BlockSpec((NHEADS, HIDW), lambda i: (0, 0)),
        ],
        out_specs=[
            pl.BlockSpec((NHEADS, BLK, TW), lambda i: (0, i, 0)),
            pl.BlockSpec((BLK, ADW), lambda i: (i, 0)),
        ],
        out_shape=[
            jax.ShapeDtypeStruct((NHEADS, NPAD, TW), jnp.float32),
            jax.ShapeDtypeStruct((NPAD, ADW), jnp.float32),
        ],
    )(x_pad, W1, a1_src, a1_dst)


def _l2_tables_body(acc_ref, b1_ref, w2_ref, a2s_ref, a2d_ref, t_ref, ad_ref):
    hs = []
    for hd in range(NHEADS):
        num = acc_ref[hd, :, 0:HIDW]
        den = acc_ref[hd, :, HIDW:HIDW + 1]
        v = num / (den + 1e-16) + b1_ref[hd, :][None, :]
        hs.append(jnp.where(v > 0, v, jnp.exp(v) - 1.0))
    h1n = jnp.concatenate(hs, axis=1)
    h2 = jnp.dot(h1n, w2_ref[...], preferred_element_type=jnp.float32)
    asv = jnp.sum(h2 * a2s_ref[0, :][None, :], axis=1, keepdims=True)
    adv = jnp.sum(h2 * a2d_ref[0, :][None, :], axis=1, keepdims=True)
    t_ref[:, 0:HIDW] = h2
    t_ref[:, HIDW:HIDW + 1] = jnp.ones((BLK, 1), jnp.float32)
    t_ref[:, HIDW + 1:HIDW + 2] = asv
    t_ref[:, HIDW + 2:TW] = jnp.zeros((BLK, TW - HIDW - 2), jnp.float32)
    ad_ref[:, 0:1] = adv
    ad_ref[:, 1:ADW] = jnp.zeros((BLK, ADW - 1), jnp.float32)


def _l2_tables(acc1, b1r, W2, a2_src, a2_dst):
    return pl.pallas_call(
        _l2_tables_body,
        grid=(NBLK,),
        in_specs=[
            pl.BlockSpec((NHEADS, BLK, TW), lambda i: (0, i, 0)),
            pl.BlockSpec((NHEADS, HIDW), lambda i: (0, 0)),
            pl.BlockSpec((NHEADS * HIDW, HIDW), lambda i: (0, 0)),
            pl.BlockSpec((1, HIDW), lambda i: (0, 0)),
            pl.BlockSpec((1, HIDW), lambda i: (0, 0)),
        ],
        out_specs=[
            pl.BlockSpec((BLK, TW), lambda i: (i, 0)),
            pl.BlockSpec((BLK, ADW), lambda i: (i, 0)),
        ],
        out_shape=[
            jax.ShapeDtypeStruct((NPAD, TW), jnp.float32),
            jax.ShapeDtypeStruct((NPAD, ADW), jnp.float32),
        ],
    )(acc1, b1r, W2, a2_src, a2_dst)


def _final_body(acc_ref, b2_ref, wout_ref, bout_ref, o_ref):
    a = acc_ref[0] + acc_ref[1]
    v = a[:, 0:HIDW] / (a[:, HIDW:HIDW + 1] + 1e-16) + b2_ref[0, :][None, :]
    h = jnp.where(v > 0, v, jnp.exp(v) - 1.0)
    o_ref[...] = jnp.dot(h, wout_ref[...],
                         preferred_element_type=jnp.float32) + bout_ref[0, :][None, :]


def _final(acc2, b2r, Wout, boutr):
    return pl.pallas_call(
        _final_body,
        grid=(NBLK,),
        in_specs=[
            pl.BlockSpec((NCORE, BLK, TW), lambda i: (0, i, 0)),
            pl.BlockSpec((1, HIDW), lambda i: (0, 0)),
            pl.BlockSpec((HIDW, 1), lambda i: (0, 0)),
            pl.BlockSpec((1, 1), lambda i: (0, 0)),
        ],
        out_specs=pl.BlockSpec((BLK, 1), lambda i: (i, 0)),
        out_shape=jax.ShapeDtypeStruct((NPAD, 1), jnp.float32),
    )(acc2, b2r, Wout, boutr)


ZROWS = 32


def _w_offsets(nb):
    offs = list(range(0, nb - 15, 16))
    if offs[-1] != nb - 16:
        offs.append(nb - 16)
    return offs


def _zero_fill(zero_v, nrows):
    zv = jnp.zeros((16,), jnp.float32)

    def zrow(j, _):
        for k in range(TW // 16):
            zero_v[j, pl.ds(k * 16, 16)] = zv
        return 0

    lax.fori_loop(0, nrows, zrow, 0)


def _scale_rows(rows, w_v, eb):
    def scale_group(g, _):
        w16 = w_v[pl.ds(g * 16, 16)]
        r0 = g * 16
        for j in range(16):
            wr = w16[j]
            for k in range(TW // 16):
                rows[r0 + j, pl.ds(k * 16, 16)] = (
                    rows[r0 + j, pl.ds(k * 16, 16)] * wr)
        return 0

    lax.fori_loop(0, eb // 16, scale_group, 0)
    rem = eb - (eb // 16) * 16
    if rem:
        r0 = (eb // 16) * 16
        w16 = w_v[pl.ds(r0, 16)]
        for j in range(rem):
            wr = w16[j]
            for k in range(TW // 16):
                rows[r0 + j, pl.ds(k * 16, 16)] = (
                    rows[r0 + j, pl.ds(k * 16, 16)] * wr)


def _pipe_step(ctx, i, b, eb, row_off, adcol, first):
    (t_hbm, ad_hbm, src_hbm, dst_hbm, acc_sp, sbuf, dbuf, scat, rbuf, abuf,
     w_v, semi, semg, sems, tile_base) = ctx
    b2 = 1 - b
    i16 = lax.iota(jnp.int32, 16)
    c129 = i16 * 0 + (HIDW + 1)
    cad = i16 * 0 + adcol
    pltpu.make_async_copy(t_hbm.at[sbuf[b]], rbuf[b], semg[b]).wait()
    pltpu.make_async_copy(ad_hbm.at[dbuf[b]], abuf[b], semg[b]).wait()
    for off in _w_offsets(eb):
        scat[b][pl.ds(off, 16)] = dbuf[b][pl.ds(off, 16)]
    base2 = tile_base + (i + 2) * eb
    pltpu.async_copy(src_hbm.at[pl.ds(base2, eb)], sbuf[b], semi[b])
    pltpu.async_copy(dst_hbm.at[pl.ds(base2, eb)], dbuf[b], semi[b])
    base1 = tile_base + (i + 1) * eb
    pltpu.make_async_copy(src_hbm.at[pl.ds(base1, eb)], sbuf[b2], semi[b2]).wait()
    pltpu.make_async_copy(dst_hbm.at[pl.ds(base1, eb)], dbuf[b2], semi[b2]).wait()
    if row_off is not None:
        for off in range(0, eb, 16):
            sbuf[b2][pl.ds(off, 16)] = sbuf[b2][pl.ds(off, 16)] + row_off
    if not first:
        pltpu.make_async_copy(rbuf[b2], acc_sp.at[scat[b2]], sems[b2]).wait()
    pltpu.async_copy(t_hbm.at[sbuf[b2]], rbuf[b2], semg[b2])
    pltpu.async_copy(ad_hbm.at[dbuf[b2]], abuf[b2], semg[b2])
    for off in _w_offsets(eb):
        rr = i16 + off
        asg = plsc.load_gather(rbuf[b], [rr, c129])
        adg = plsc.load_gather(abuf[b], [rr, cad])
        e = asg + adg
        e = jnp.maximum(e, e * 0.2)
        w_v[pl.ds(off, 16)] = jnp.exp(e)
    _scale_rows(rbuf[b], w_v, eb)
    pltpu.async_copy(rbuf[b], acc_sp.at[scat[b]], sems[b], add=True)


def _edge_pass(ctx, nb, eb, row_off, adcol):
    (t_hbm, ad_hbm, src_hbm, dst_hbm, acc_sp, sbuf, dbuf, scat, rbuf, abuf,
     w_v, semi, semg, sems, tile_base) = ctx
    assert (nb - 1) % 2 == 0
    for j in range(2):
        base = tile_base + j * eb
        pltpu.async_copy(src_hbm.at[pl.ds(base, eb)], sbuf[j], semi[j])
        pltpu.async_copy(dst_hbm.at[pl.ds(base, eb)], dbuf[j], semi[j])
    pltpu.make_async_copy(src_hbm.at[pl.ds(tile_base, eb)], sbuf[0], semi[0]).wait()
    pltpu.make_async_copy(dst_hbm.at[pl.ds(tile_base, eb)], dbuf[0], semi[0]).wait()
    if row_off is not None:
        for off in range(0, eb, 16):
            sbuf[0][pl.ds(off, 16)] = sbuf[0][pl.ds(off, 16)] + row_off
    pltpu.async_copy(t_hbm.at[sbuf[0]], rbuf[0], semg[0])
    pltpu.async_copy(ad_hbm.at[dbuf[0]], abuf[0], semg[0])
    _pipe_step(ctx, 0, 0, eb, row_off, adcol, True)

    def gbody(g, _):
        _pipe_step(ctx, 2 * g + 1, 1, eb, row_off, adcol, False)
        _pipe_step(ctx, 2 * g + 2, 0, eb, row_off, adcol, False)
        return 0

    lax.fori_loop(0, (nb - 1) // 2, gbody, 0)
    bg = nb % 2
    bs = (nb - 1) % 2
    pltpu.make_async_copy(t_hbm.at[sbuf[bg]], rbuf[bg], semg[bg]).wait()
    pltpu.make_async_copy(ad_hbm.at[dbuf[bg]], abuf[bg], semg[bg]).wait()
    pltpu.make_async_copy(rbuf[bs], acc_sp.at[scat[bs]], sems[bs]).wait()
    base = tile_base + (nb + 1) * eb
    pltpu.make_async_copy(src_hbm.at[pl.ds(base, eb)], sbuf[bs], semi[bs]).wait()
    pltpu.make_async_copy(dst_hbm.at[pl.ds(base, eb)], dbuf[bs], semi[bs]).wait()


def _acc_zero(acc_sp, zero_v, s):
    def zacc(j, _):
        pltpu.sync_copy(zero_v,
                        acc_sp.at[pl.ds(s * ROWS_PER_TILE + j * ZROWS, ZROWS)])
        return 0
    lax.fori_loop(0, ROWS_PER_TILE // ZROWS, zacc, 0)


def _acc_readout(acc_sp, stage_v, out_hbm, s, out_row_base):
    def rdout(j, _):
        rs = s * ROWS_PER_TILE + j * ZROWS
        pltpu.sync_copy(acc_sp.at[pl.ds(rs, ZROWS)], stage_v)
        pltpu.sync_copy(stage_v, out_hbm.at[pl.ds(out_row_base + rs, ZROWS)])
        return 0
    lax.fori_loop(0, ROWS_PER_TILE // ZROWS, rdout, 0)


def _sc_scratch(eb, wlen):
    return [
        pltpu.VMEM_SHARED((NPAD, TW), jnp.float32),
        pltpu.VMEM((eb,), jnp.int32),
        pltpu.VMEM((eb,), jnp.int32),
        pltpu.VMEM((eb,), jnp.int32),
        pltpu.VMEM((eb,), jnp.int32),
        pltpu.VMEM((eb,), jnp.int32),
        pltpu.VMEM((eb,), jnp.int32),
        pltpu.VMEM((eb, TW), jnp.float32),
        pltpu.VMEM((eb, TW), jnp.float32),
        pltpu.VMEM((eb, ADW), jnp.float32),
        pltpu.VMEM((eb, ADW), jnp.float32),
        pltpu.VMEM((wlen,), jnp.float32),
        pltpu.VMEM((ZROWS, TW), jnp.float32),
        pltpu.VMEM((ZROWS, TW), jnp.float32),
        pltpu.SemaphoreType.DMA,
        pltpu.SemaphoreType.DMA,
        pltpu.SemaphoreType.DMA,
        pltpu.SemaphoreType.DMA,
        pltpu.SemaphoreType.DMA,
        pltpu.SemaphoreType.DMA,
    ]


def _sc_layer1(tflat, adtab, src, dst):
    mesh = plsc.VectorSubcoreMesh(core_axis_name="c", subcore_axis_name="s")

    @functools.partial(
        pl.kernel,
        out_type=jax.ShapeDtypeStruct((NHEADS * NPAD, TW), jnp.float32),
        mesh=mesh,
        compiler_params=pltpu.CompilerParams(use_tc_tiling_on_sc=False,
                                             needs_layout_passes=False),
        scratch_types=_sc_scratch(EB1, EB1),
    )
    def body(t_hbm, ad_hbm, src_hbm, dst_hbm, out_hbm, acc_sp, s0, s1, d0, d1,
             x0, x1, r0, r1, a0, a1, w_v, zero_v, stage_v,
             si0, si1, sg0, sg1, ss0, ss1):
        c = lax.axis_index("c")
        s = lax.axis_index("s")
        _zero_fill(zero_v, ZROWS)
        ctx = (t_hbm, ad_hbm, src_hbm, dst_hbm, acc_sp, (s0, s1), (d0, d1),
               (x0, x1), (r0, r1), (a0, a1), w_v, (si0, si1), (sg0, sg1),
               (ss0, ss1), s * (EE // NTILE))
        for p in range(2):
            hd = 2 * c + p
            _acc_zero(acc_sp, zero_v, s)
            plsc.subcore_barrier()
            _edge_pass(ctx, NB1, EB1, hd * NPAD, hd)
            plsc.subcore_barrier()
            _acc_readout(acc_sp, stage_v, out_hbm, s, hd * NPAD)
            plsc.subcore_barrier()

    return body(tflat, adtab, src, dst)


def _sc_layer2(tbl, adtab, src, dst):
    mesh = plsc.VectorSubcoreMesh(core_axis_name="c", subcore_axis_name="s")

    @functools.partial(
        pl.kernel,
        out_type=jax.ShapeDtypeStruct((NCORE * NPAD, TW), jnp.float32),
        mesh=mesh,
        compiler_params=pltpu.CompilerParams(use_tc_tiling_on_sc=False,
                                             needs_layout_passes=False),
        scratch_types=_sc_scratch(EB2, 48),
    )
    def body(t_hbm, ad_hbm, src_hbm, dst_hbm, out_hbm, acc_sp, s0, s1, d0, d1,
             x0, x1, r0, r1, a0, a1, w_v, zero_v, stage_v,
             si0, si1, sg0, sg1, ss0, ss1):
        c = lax.axis_index("c")
        s = lax.axis_index("s")
        wid = c * NTILE + s
        _zero_fill(zero_v, ZROWS)
        _acc_zero(acc_sp, zero_v, s)
        plsc.subcore_barrier()
        ctx = (t_hbm, ad_hbm, src_hbm, dst_hbm, acc_sp, (s0, s1), (d0, d1),
               (x0, x1), (r0, r1), (a0, a1), w_v, (si0, si1), (sg0, sg1),
               (ss0, ss1), wid * (EE // (NTILE * NCORE)))
        _edge_pass(ctx, NB2, EB2, None, 0)
        plsc.subcore_barrier()
        _acc_readout(acc_sp, stage_v, out_hbm, s, c * NPAD)

    return body(tbl, adtab, src, dst)


def kernel(x, edge_index, W1, a1_src, a1_dst, b1, W2, a2_src, a2_dst, b2,
           Wout, bout):
    x_pad = jnp.pad(x, ((0, NPAD - NN), (0, 0)))
    src = jnp.pad(edge_index[0], (0, EIDX_PAD - EE))
    dst = jnp.pad(edge_index[1], (0, EIDX_PAD - EE))

    table1, adtab1 = _l1_tables(x_pad, W1, a1_src, a1_dst)
    acc1 = _sc_layer1(table1.reshape(NHEADS * NPAD, TW), adtab1, src, dst)
    acc1 = acc1.reshape(NHEADS, NPAD, TW)

    table2, adtab2 = _l2_tables(acc1, b1.reshape(NHEADS, HIDW), W2,
                                a2_src, a2_dst)
    acc2 = _sc_layer2(table2, adtab2, src, dst)
    acc2 = acc2.reshape(NCORE, NPAD, TW)

    y = _final(acc2, b2.reshape(1, HIDW), Wout, bout.reshape(1, 1))
    return y[:NN]
